# baseline (speedup 1.0000x reference)
import jax
import jax.numpy as jnp
from jax import lax
from jax.experimental import pallas as pl
from jax.experimental.pallas import tpu as pltpu
from jax.experimental.pallas import tpu_sc as plsc

B = 4096
L = 200
D = 32
LANES = 16

NC = 2
NS = 16
NW = NC * NS
BPW = B // NW
CL = 8
N_CHUNKS = L // CL
TILE = 8 * 128
RP = D + 1


def _emb_body(table_hbm, xt_hbm, pe_hbm, scale_hbm, out_hbm,
              idx_v, rows_v, rows2_v, q_v, pe_v, scale_v, sem, sem_out):
    w = lax.axis_index("s") * NC + lax.axis_index("c")

    pltpu.sync_copy(pe_hbm.at[pl.ds(0, L)], pe_v)
    pltpu.sync_copy(scale_hbm, scale_v)
    sv = scale_v[...]
    iota = lax.iota(jnp.int32, LANES)

    def chunk_body(c, carry):
        l0 = c * CL
        pltpu.sync_copy(xt_hbm.at[pl.ds(l0, CL), pl.ds(w * BPW, BPW)], idx_v)
        copies = [
            pltpu.async_copy(
                table_hbm.at[idx_v.at[li]],
                rows_v.at[pl.ds(li * BPW, BPW)],
                sem,
            )
            for li in range(CL)
        ]
        for cp in copies:
            cp.wait()

        for li in range(CL):
            l = l0 + li
            pe_lo = pe_v[l, pl.ds(0, LANES)]
            pe_hi = pe_v[l, pl.ds(LANES, LANES)]

            @plsc.parallel_loop(0, BPW // 8, unroll=2)
            def bl_body(bl8, li=li, pe_lo=pe_lo, pe_hi=pe_hi):
                for s in range(8):
                    r = li * BPW + bl8 * 8 + s
                    rows2_v[r, pl.ds(0, LANES)] = (
                        rows_v[r, pl.ds(0, LANES)] * sv + pe_lo)
                    rows2_v[r, pl.ds(LANES, LANES)] = (
                        rows_v[r, pl.ds(LANES, LANES)] * sv + pe_hi)

        @plsc.parallel_loop(0, CL * (BPW // LANES), unroll=2)
        def blk_body(i):
            li = i // (BPW // LANES)
            blk = i % (BPW // LANES)
            row_idx = iota + (li * BPW + blk * LANES)
            qcol = blk * LANES
            for d in range(D):
                col = jnp.full((LANES,), d, jnp.int32)
                val = plsc.load_gather(rows2_v, [row_idx, col])
                q_v[li * 4 + d // 8,
                    pl.ds((d % 8) * 128 + qcol, LANES)] = val

        pltpu.async_copy(
            q_v,
            out_hbm.at[pl.ds(l0 * 4, CL * 4), pl.ds(w * TILE, TILE)],
            sem_out,
        ).wait()
        return carry

    lax.fori_loop(0, N_CHUNKS, chunk_body, 0)


def kernel(x, table, pe, scale):
    xt = jnp.asarray(x, jnp.int32).T
    scale_v = jnp.broadcast_to(scale.astype(jnp.float32), (LANES,))
    mesh = plsc.VectorSubcoreMesh(core_axis_name="c", subcore_axis_name="s")
    q = pl.kernel(
        _emb_body,
        out_type=jax.ShapeDtypeStruct((L * (D // 8), NW * TILE), jnp.float32),
        mesh=mesh,
        compiler_params=pltpu.CompilerParams(
            use_tc_tiling_on_sc=False, needs_layout_passes=False),
        scratch_types=[
            pltpu.VMEM((CL, BPW), jnp.int32),
            pltpu.VMEM((CL * BPW, D), jnp.float32),
            pltpu.VMEM((CL * BPW, RP), jnp.float32),
            pltpu.VMEM((CL * 4, TILE), jnp.float32),
            pltpu.VMEM((L, D), jnp.float32),
            pltpu.VMEM((LANES,), jnp.float32),
            pltpu.SemaphoreType.DMA,
            pltpu.SemaphoreType.DMA,
        ],
    )(table, xt, pe, scale_v)
    out = (
        q.reshape(L, D // 8, NW, 8, BPW)
        .transpose(2, 4, 0, 1, 3)
        .reshape(B, L, D)
    )
    return out
